# baseline (speedup 1.0000x reference)
import jax
import jax.numpy as jnp
from jax.experimental import pallas as pl
from jax.experimental.pallas import tpu as pltpu

B = 16384
DIM_OBS = 128
HID = 756
HID_PAD = 768
ACTION_DIM = 16
BM = 1024


def _mlp_block(state_ref, w1_ref, b1_ref, w2_ref, b2_ref, w3_ref, b3_ref,
               out_ref):
    h = jnp.dot(state_ref[:], w1_ref[:], preferred_element_type=jnp.float32)
    h = jnp.maximum(h.astype(jnp.bfloat16) + b1_ref[:], 0)
    h = jnp.dot(h, w2_ref[:], preferred_element_type=jnp.float32)
    h = jnp.maximum(h.astype(jnp.bfloat16) + b2_ref[:], 0)
    h = jnp.dot(h, w3_ref[:], preferred_element_type=jnp.float32)
    out_ref[:] = h + b3_ref[:]


@jax.jit
def kernel(state, W1, b1, W2, b2, W3, b3):
    pad_h = HID_PAD - HID
    state = state.astype(jnp.bfloat16)
    w1t = jnp.pad(W1.T, ((0, 0), (0, pad_h))).astype(jnp.bfloat16)
    w2t = jnp.pad(W2.T, ((0, pad_h), (0, pad_h))).astype(jnp.bfloat16)
    w3t = jnp.pad(W3.T, ((0, pad_h), (0, 0))).astype(jnp.bfloat16)
    b1p = jnp.pad(b1, (0, pad_h)).astype(jnp.bfloat16).reshape(1, HID_PAD)
    b2p = jnp.pad(b2, (0, pad_h)).astype(jnp.bfloat16).reshape(1, HID_PAD)
    b3p = b3.reshape(1, ACTION_DIM)

    grid = (B // BM,)
    fixed = lambda i: (0, 0)
    return pl.pallas_call(
        _mlp_block,
        grid=grid,
        in_specs=[
            pl.BlockSpec((BM, DIM_OBS), lambda i: (i, 0)),
            pl.BlockSpec((DIM_OBS, HID_PAD), fixed),
            pl.BlockSpec((1, HID_PAD), fixed),
            pl.BlockSpec((HID_PAD, HID_PAD), fixed),
            pl.BlockSpec((1, HID_PAD), fixed),
            pl.BlockSpec((HID_PAD, ACTION_DIM), fixed),
            pl.BlockSpec((1, ACTION_DIM), fixed),
        ],
        out_specs=pl.BlockSpec((BM, ACTION_DIM), lambda i: (i, 0)),
        out_shape=jax.ShapeDtypeStruct((B, ACTION_DIM), jnp.float32),
        compiler_params=pltpu.CompilerParams(
            dimension_semantics=("parallel",),
        ),
    )(state, w1t, b1p, w2t, b2p, w3t, b3p)


# speedup vs baseline: 1.2251x; 1.2251x over previous
import jax
import jax.numpy as jnp
from jax.experimental import pallas as pl
from jax.experimental.pallas import tpu as pltpu

B = 16384
DIM_OBS = 128
HID = 756
ACTION_DIM = 16
BM = 1024

_DNUMS = (((1,), (1,)), ((), ()))


def _mlp_block(state_ref, w1_ref, b1_ref, w2_ref, b2_ref, w3_ref, b3_ref,
               out_ref):
    x = state_ref[:].astype(jnp.bfloat16)
    w1 = w1_ref[:].astype(jnp.bfloat16)
    h = jax.lax.dot_general(x, w1, _DNUMS, preferred_element_type=jnp.float32)
    h = jnp.maximum(h + b1_ref[:], 0.0).astype(jnp.bfloat16)
    w2 = w2_ref[:].astype(jnp.bfloat16)
    h = jax.lax.dot_general(h, w2, _DNUMS, preferred_element_type=jnp.float32)
    h = jnp.maximum(h + b2_ref[:], 0.0).astype(jnp.bfloat16)
    w3 = w3_ref[:].astype(jnp.bfloat16)
    h = jax.lax.dot_general(h, w3, _DNUMS, preferred_element_type=jnp.float32)
    out_ref[:] = h + b3_ref[:]


@jax.jit
def kernel(state, W1, b1, W2, b2, W3, b3):
    grid = (B // BM,)
    fixed = lambda i: (0, 0)
    return pl.pallas_call(
        _mlp_block,
        grid=grid,
        in_specs=[
            pl.BlockSpec((BM, DIM_OBS), lambda i: (i, 0)),
            pl.BlockSpec((HID, DIM_OBS), fixed),
            pl.BlockSpec((1, HID), fixed),
            pl.BlockSpec((HID, HID), fixed),
            pl.BlockSpec((1, HID), fixed),
            pl.BlockSpec((ACTION_DIM, HID), fixed),
            pl.BlockSpec((1, ACTION_DIM), fixed),
        ],
        out_specs=pl.BlockSpec((BM, ACTION_DIM), lambda i: (i, 0)),
        out_shape=jax.ShapeDtypeStruct((B, ACTION_DIM), jnp.float32),
        compiler_params=pltpu.CompilerParams(
            dimension_semantics=("parallel",),
        ),
    )(state, W1, b1.reshape(1, HID), W2, b2.reshape(1, HID), W3,
      b3.reshape(1, ACTION_DIM))


# speedup vs baseline: 1.2886x; 1.0519x over previous
import jax
import jax.numpy as jnp
from jax.experimental import pallas as pl
from jax.experimental.pallas import tpu as pltpu

B = 16384
DIM_OBS = 128
HID = 756
ACTION_DIM = 16
BM = 2048

_DNUMS = (((1,), (1,)), ((), ()))


def _mlp_block(state_ref, w1_ref, b1_ref, w2_ref, b2_ref, w3_ref, b3_ref,
               out_ref):
    x = state_ref[:].astype(jnp.bfloat16)
    w1 = w1_ref[:].astype(jnp.bfloat16)
    h = jax.lax.dot_general(x, w1, _DNUMS, preferred_element_type=jnp.float32)
    h = jnp.maximum(h + b1_ref[:], 0.0).astype(jnp.bfloat16)
    w2 = w2_ref[:].astype(jnp.bfloat16)
    h = jax.lax.dot_general(h, w2, _DNUMS, preferred_element_type=jnp.float32)
    h = jnp.maximum(h + b2_ref[:], 0.0).astype(jnp.bfloat16)
    w3 = w3_ref[:].astype(jnp.bfloat16)
    h = jax.lax.dot_general(h, w3, _DNUMS, preferred_element_type=jnp.float32)
    out_ref[:] = h + b3_ref[:]


@jax.jit
def kernel(state, W1, b1, W2, b2, W3, b3):
    grid = (B // BM,)
    fixed = lambda i: (0, 0)
    return pl.pallas_call(
        _mlp_block,
        grid=grid,
        in_specs=[
            pl.BlockSpec((BM, DIM_OBS), lambda i: (i, 0)),
            pl.BlockSpec((HID, DIM_OBS), fixed),
            pl.BlockSpec((1, HID), fixed),
            pl.BlockSpec((HID, HID), fixed),
            pl.BlockSpec((1, HID), fixed),
            pl.BlockSpec((ACTION_DIM, HID), fixed),
            pl.BlockSpec((1, ACTION_DIM), fixed),
        ],
        out_specs=pl.BlockSpec((BM, ACTION_DIM), lambda i: (i, 0)),
        out_shape=jax.ShapeDtypeStruct((B, ACTION_DIM), jnp.float32),
        compiler_params=pltpu.CompilerParams(
            dimension_semantics=("parallel",),
        ),
    )(state, W1, b1.reshape(1, HID), W2, b2.reshape(1, HID), W3,
      b3.reshape(1, ACTION_DIM))


# speedup vs baseline: 1.3033x; 1.0114x over previous
import jax
import jax.numpy as jnp
from jax.experimental import pallas as pl
from jax.experimental.pallas import tpu as pltpu

B = 16384
DIM_OBS = 128
HID = 756
ACTION_DIM = 16
BM = 4096

_DNUMS = (((1,), (1,)), ((), ()))


def _mlp_block(state_ref, w1_ref, b1_ref, w2_ref, b2_ref, w3_ref, b3_ref,
               out_ref):
    x = state_ref[:].astype(jnp.bfloat16)
    w1 = w1_ref[:].astype(jnp.bfloat16)
    h = jax.lax.dot_general(x, w1, _DNUMS, preferred_element_type=jnp.float32)
    h = jnp.maximum(h + b1_ref[:], 0.0).astype(jnp.bfloat16)
    w2 = w2_ref[:].astype(jnp.bfloat16)
    h = jax.lax.dot_general(h, w2, _DNUMS, preferred_element_type=jnp.float32)
    h = jnp.maximum(h + b2_ref[:], 0.0).astype(jnp.bfloat16)
    w3 = w3_ref[:].astype(jnp.bfloat16)
    h = jax.lax.dot_general(h, w3, _DNUMS, preferred_element_type=jnp.float32)
    out_ref[:] = h + b3_ref[:]


@jax.jit
def kernel(state, W1, b1, W2, b2, W3, b3):
    grid = (B // BM,)
    fixed = lambda i: (0, 0)
    return pl.pallas_call(
        _mlp_block,
        grid=grid,
        in_specs=[
            pl.BlockSpec((BM, DIM_OBS), lambda i: (i, 0)),
            pl.BlockSpec((HID, DIM_OBS), fixed),
            pl.BlockSpec((1, HID), fixed),
            pl.BlockSpec((HID, HID), fixed),
            pl.BlockSpec((1, HID), fixed),
            pl.BlockSpec((ACTION_DIM, HID), fixed),
            pl.BlockSpec((1, ACTION_DIM), fixed),
        ],
        out_specs=pl.BlockSpec((BM, ACTION_DIM), lambda i: (i, 0)),
        out_shape=jax.ShapeDtypeStruct((B, ACTION_DIM), jnp.float32),
        compiler_params=pltpu.CompilerParams(
            dimension_semantics=("parallel",),
        ),
    )(state, W1, b1.reshape(1, HID), W2, b2.reshape(1, HID), W3,
      b3.reshape(1, ACTION_DIM))
